# baseline (speedup 1.0000x reference)
import functools

import jax
import jax.numpy as jnp
from jax import lax
from jax.experimental import pallas as pl
from jax.experimental.pallas import tpu as pltpu
from jax.experimental.pallas import tpu_sc as plsc

NUM_Q = 100000
EMB = 64

NC = 2
NS = 16
L = 16
NW = NC * NS

CH = 512
IB = 128
KSUB = CH // IB


def _make_gather(B: int):
    assert B % (NW * CH) == 0
    b_per_w = B // NW
    rows_per_w = b_per_w // IB
    n_chunks = b_per_w // CH
    assert n_chunks % 2 == 0
    mesh = plsc.VectorSubcoreMesh(core_axis_name="c", subcore_axis_name="s")

    @functools.partial(
        pl.kernel,
        mesh=mesh,
        compiler_params=pltpu.CompilerParams(use_tc_tiling_on_sc=False),
        out_type=jax.ShapeDtypeStruct((B, EMB), jnp.float32),
        scratch_types=[
            pltpu.VMEM((rows_per_w, IB), jnp.int32),
            pltpu.VMEM((rows_per_w, IB), jnp.int32),
            pltpu.VMEM((CH, EMB), jnp.float32),
            pltpu.VMEM((CH, EMB), jnp.float32),
            pltpu.SemaphoreType.DMA,
            pltpu.SemaphoreType.DMA,
            pltpu.SemaphoreType.DMA,
            pltpu.SemaphoreType.DMA,
        ],
    )
    def gather_kernel(q_hbm, r_hbm, table_hbm, out_hbm,
                      idxv, rv, rows0, rows1, gsem0, gsem1, osem0, osem1):
        wid = lax.axis_index("s") * NC + lax.axis_index("c")
        base = wid * b_per_w

        pltpu.sync_copy(q_hbm.at[pl.ds(wid * rows_per_w, rows_per_w)], idxv)
        pltpu.sync_copy(r_hbm.at[pl.ds(wid * rows_per_w, rows_per_w)], rv)

        @pl.loop(0, rows_per_w)
        def _compute_idx(t):
            for s in range(IB // L):
                sl = pl.ds(s * L, L)
                idxv[t, sl] = idxv[t, sl] + NUM_Q * rv[t, sl]

        def fire(c, rows, gsem):
            for j in range(KSUB):
                pltpu.async_copy(
                    table_hbm.at[idxv.at[c * KSUB + j]],
                    rows.at[pl.ds(j * IB, IB)],
                    gsem,
                )

        def drain_gather(rows, gsem):
            pltpu.make_async_copy(out_hbm.at[pl.ds(0, CH)], rows, gsem).wait()

        def store(c, rows, osem):
            return pltpu.async_copy(rows, out_hbm.at[pl.ds(base + c * CH, CH)], osem)

        def drain_store(rows, osem):
            pltpu.make_async_copy(rows, out_hbm.at[pl.ds(0, CH)], osem).wait()

        fire(0, rows0, gsem0)

        @pl.loop(0, n_chunks // 2)
        def _pair(i):
            c0 = 2 * i
            @pl.when(i > 0)
            def _():
                drain_store(rows1, osem1)

            fire(c0 + 1, rows1, gsem1)
            drain_gather(rows0, gsem0)
            st0 = store(c0, rows0, osem0)
            drain_gather(rows1, gsem1)
            st0.wait()

            @pl.when(i < n_chunks // 2 - 1)
            def _():
                fire(c0 + 2, rows0, gsem0)

            store(c0 + 1, rows1, osem1)

        drain_store(rows1, osem1)

    return gather_kernel


def kernel(q, r, interaction_emb):
    shape = q.shape
    B = q.size
    qf = q.reshape(B // IB, IB).astype(jnp.int32)
    rf = r.reshape(B // IB, IB).astype(jnp.int32)
    out = _make_gather(B)(qf, rf, interaction_emb)
    return out.reshape(*shape, EMB)
